# Initial kernel scaffold; baseline (speedup 1.0000x reference)
import jax
import jax.numpy as jnp
from jax.experimental import pallas as pl


def kernel(x, edge_index, edge_weight, W_self, W_neigh, b, gates):
    raise NotImplementedError("write your pallas kernel here")


# speedup vs baseline: 4.5051x; 4.5051x over previous
import functools

import jax
import jax.numpy as jnp
from jax import lax
from jax.experimental import pallas as pl
from jax.experimental.pallas import tpu as pltpu
from jax.experimental.pallas import tpu_sc as plsc

N = 10000
D = 128
E = 320000
L = 4

LANES = 16
NC = 2
NS = 16
NW = NC * NS
B = 128
ROWS = E // B
RPW = (ROWS + NW - 1) // NW
NP = 10240
NPT = NP // NS


def _seg_body(x_hbm, src_hbm, dst_hbm, ew_hbm, out_hbm,
              src_v, dst_v, ew_v, rows_v, acc_sh, sem):
    cid = lax.axis_index("c")
    sid = lax.axis_index("s")
    wid = sid * NC + cid

    zero = jnp.zeros((LANES,), jnp.float32)

    def _zrow(r, c):
        for j in range(D // LANES):
            rows_v[r, pl.ds(LANES * j, LANES)] = zero
        return c

    lax.fori_loop(0, B, _zrow, 0)
    base = sid * NPT
    for k in range(NPT // B):
        pltpu.sync_copy(rows_v, acc_sh.at[pl.ds(base + B * k, B)])
    plsc.subcore_barrier()

    def _batch(t, c):
        r = wid + t * NW

        @pl.when(r < ROWS)
        def _():
            pltpu.sync_copy(src_hbm.at[r], src_v)
            pltpu.sync_copy(dst_hbm.at[r], dst_v)
            pltpu.sync_copy(ew_hbm.at[r], ew_v)
            pltpu.async_copy(x_hbm.at[src_v], rows_v, sem).wait()

            def _grp(gi, cc):
                wv = ew_v[pl.ds(LANES * gi, LANES)]
                for rr in range(LANES):
                    e = gi * LANES + rr
                    w = jnp.full((LANES,), wv[rr], jnp.float32)
                    for j in range(D // LANES):
                        v = rows_v[e, pl.ds(LANES * j, LANES)]
                        v = jnp.maximum(v, 0.01 * v) * w
                        rows_v[e, pl.ds(LANES * j, LANES)] = v
                return cc

            lax.fori_loop(0, B // LANES, _grp, 0)
            pltpu.sync_copy(rows_v, acc_sh.at[dst_v], add=True)

        return c

    lax.fori_loop(0, RPW, _batch, 0)
    plsc.subcore_barrier()
    pltpu.sync_copy(acc_sh.at[pl.ds(base, NPT)],
                    out_hbm.at[cid, pl.ds(base, NPT)])


_seg = pl.kernel(
    _seg_body,
    out_type=jax.ShapeDtypeStruct((NC, NP, D), jnp.float32),
    mesh=plsc.VectorSubcoreMesh(core_axis_name="c", subcore_axis_name="s",
                                num_cores=NC, num_subcores=NS),
    scratch_types=[
        pltpu.VMEM((B,), jnp.int32),
        pltpu.VMEM((B,), jnp.int32),
        pltpu.VMEM((B,), jnp.float32),
        pltpu.VMEM((B, D), jnp.float32),
        pltpu.VMEM_SHARED((NP, D), jnp.float32),
        pltpu.SemaphoreType.DMA,
    ],
)

BN = 1000


def _tc_body(x_ref, p_ref, ws_ref, wn_ref, b_ref, g_ref, o_ref):
    x = x_ref[...]
    h = jnp.maximum(x, 0.01 * x)
    agg = p_ref[0] + p_ref[1]
    o_ref[...] = (jnp.dot(h, ws_ref[...], preferred_element_type=jnp.float32)
                  + jnp.dot(agg, wn_ref[...], preferred_element_type=jnp.float32)
                  + b_ref[...] + g_ref[0] * x)


_tc = pl.pallas_call(
    _tc_body,
    grid=(N // BN,),
    in_specs=[
        pl.BlockSpec((BN, D), lambda i: (i, 0)),
        pl.BlockSpec((NC, BN, D), lambda i: (0, i, 0)),
        pl.BlockSpec((D, D), lambda i: (0, 0)),
        pl.BlockSpec((D, D), lambda i: (0, 0)),
        pl.BlockSpec((1, D), lambda i: (0, 0)),
        pl.BlockSpec((1, 1), lambda i: (0, 0)),
    ],
    out_specs=pl.BlockSpec((BN, D), lambda i: (i, 0)),
    out_shape=jax.ShapeDtypeStruct((N, D), jnp.float32),
)


def kernel(x, edge_index, edge_weight, W_self, W_neigh, b, gates):
    g = jax.nn.sigmoid(gates)
    src2 = edge_index[0].reshape(ROWS, B)
    dst2 = edge_index[1].reshape(ROWS, B)
    ew2 = edge_weight.reshape(ROWS, B)
    for i in range(L):
        part = _seg(x, src2, dst2, ew2)
        gi = g[i]
        x = _tc(x, part,
                (1.0 - gi) * W_self[i], (1.0 - gi) * W_neigh[i],
                ((1.0 - gi) * b[i]).reshape(1, D), gi.reshape(1, 1))
    return x
